# baseline (speedup 1.0000x reference)
import functools

import jax
import jax.numpy as jnp
from jax import lax
from jax.experimental import pallas as pl
from jax.experimental.pallas import tpu as pltpu
from jax.experimental.pallas import tpu_sc as plsc

N = 10000
E = 320000
NH = 128
EPS_COS = 1e-8

NC, NS = 2, 16
NW = NC * NS

EREAL = E // NS
EPW = 20480
S_CHUNK = 128
S_NCH = EPW // S_CHUNK
NP = 10240
ROWS_PW = NP // NS
ZR = 160
HW = 64


def _vtake(vec, idx):
    dn = lax.GatherDimensionNumbers(offset_dims=(), collapsed_slice_dims=(0,),
                                    start_index_map=(0,))
    return lax.gather(vec, idx[:, None], dn, slice_sizes=(1,),
                      mode=lax.GatherScatterMode.PROMISE_IN_BOUNDS)


def _scale_chunk(rows_ref, de_ref):
    def group(g, _):
        wv16 = lax.bitcast_convert_type(de_ref[2, pl.ds(g * 16, 16)],
                                        jnp.float32)
        for i in range(16):
            wv = _vtake(wv16, jnp.full((16,), i, jnp.int32))
            r = g * 16 + i
            for j in range(HW // 16):
                sl = pl.ds(j * 16, 16)
                rows_ref[r, sl] = rows_ref[r, sl] * wv
        return 0

    lax.fori_loop(0, S_CHUNK // 16, group, 0)


NBUF = 5


def _sc_spmm_body(h_hbm, ed_hbm, out_hbm,
                  idx_bufs, dc_bufs, de_bufs, rows_bufs,
                  zbuf, acc, gsems, ssems, esems):
    c = lax.axis_index("c")
    s = lax.axis_index("s")

    for hh in range(2):
        def zrow(r, _):
            for j in range(HW // 16):
                zbuf[r, pl.ds(j * 16, 16)] = jnp.zeros((16,), jnp.float32)
            return 0

        lax.fori_loop(0, ZR, zrow, 0, unroll=4)
        for k in range(ROWS_PW // ZR):
            pltpu.sync_copy(zbuf, acc.at[pl.ds(s * ROWS_PW + k * ZR, ZR)])
        plsc.subcore_barrier()

        rbase = 2 * c * N + hh

        def fire_echunk(t, b):
            pltpu.async_copy(ed_hbm.at[s * S_NCH + t], de_bufs[b], esems[b])

        def prep_and_fire(t, b):
            pltpu.make_async_copy(ed_hbm.at[0], de_bufs[b], esems[b]).wait()
            for g in range(S_CHUNK // 16):
                sl = pl.ds(g * 16, 16)
                idx_bufs[b][sl] = de_bufs[b][1, sl] * 2 + rbase
                dc_bufs[b][sl] = de_bufs[b][0, sl]
            pltpu.async_copy(h_hbm.at[idx_bufs[b]], rows_bufs[b], gsems[b])

        fire_echunk(0, 0)
        fire_echunk(1, 1)
        prep_and_fire(0, 0)
        fire_echunk(2, 2)
        prep_and_fire(1, 1)

        def quint(q, _):
            v0 = q * NBUF
            for b in range(NBUF):
                v = v0 + b
                pltpu.make_async_copy(h_hbm.at[idx_bufs[b]], rows_bufs[b],
                                      gsems[b]).wait()
                _scale_chunk(rows_bufs[b], de_bufs[b])
                pltpu.async_copy(rows_bufs[b], acc.at[dc_bufs[b]], ssems[b],
                                 add=True)
                bp = (b + 2) % NBUF

                bq = (b + 3) % NBUF

                @pl.when(v + 3 < S_NCH)
                def _():
                    fire_echunk(v + 3, bq)

                @pl.when(v + 2 < S_NCH)
                def _():
                    @pl.when(v >= 3)
                    def _():
                        pltpu.make_async_copy(rows_bufs[bp], acc.at[dc_bufs[bp]],
                                              ssems[bp]).wait()

                    prep_and_fire(v + 2, bp)

            return 0

        lax.fori_loop(0, S_NCH // NBUF, quint, 0)

        for b in range(NBUF):
            pltpu.make_async_copy(rows_bufs[b], acc.at[dc_bufs[b]],
                                  ssems[b]).wait()

        plsc.subcore_barrier()
        obase = (2 * c + hh) * NP
        for k in range(ROWS_PW // ZR):
            r0 = s * ROWS_PW + k * ZR
            pltpu.sync_copy(acc.at[pl.ds(r0, ZR)],
                            out_hbm.at[pl.ds(obase + r0, ZR)])
        plsc.subcore_barrier()


def _sc_spmm(h64, edata):
    mesh = plsc.VectorSubcoreMesh(core_axis_name="c", subcore_axis_name="s")
    return pl.kernel(
        _sc_spmm_body,
        out_type=jax.ShapeDtypeStruct((4 * NP, HW), jnp.float32),
        mesh=mesh,
        compiler_params=pltpu.CompilerParams(use_tc_tiling_on_sc=False),
        scratch_types=[
            [pltpu.VMEM((S_CHUNK,), jnp.int32) for _ in range(NBUF)],
            [pltpu.VMEM((S_CHUNK,), jnp.int32) for _ in range(NBUF)],
            [pltpu.VMEM((3, S_CHUNK), jnp.int32) for _ in range(NBUF)],
            [pltpu.VMEM((S_CHUNK, HW), jnp.float32) for _ in range(NBUF)],
            pltpu.VMEM((ZR, HW), jnp.float32),
            pltpu.VMEM_SHARED((NP, HW), jnp.float32),
            [pltpu.SemaphoreType.DMA for _ in range(NBUF)],
            [pltpu.SemaphoreType.DMA for _ in range(NBUF)],
            [pltpu.SemaphoreType.DMA for _ in range(NBUF)],
        ],
    )(h64, edata)


G_TOT = 50176
G_PER_W = G_TOT // NW
G_CHUNK = 112
G_NCH = G_PER_W // G_CHUNK


def _sc_gather_body(table_hbm, idx_hbm, out_hbm, idx_v, rows_v, sem):
    wid = lax.axis_index("s") * NC + lax.axis_index("c")
    base0 = wid * G_PER_W

    def chunk(t, _):
        base = base0 + t * G_CHUNK
        pltpu.sync_copy(idx_hbm.at[pl.ds(base, G_CHUNK)], idx_v)
        pltpu.async_copy(table_hbm.at[idx_v], rows_v, sem).wait()
        pltpu.sync_copy(rows_v, out_hbm.at[pl.ds(base, G_CHUNK)])
        return 0

    lax.fori_loop(0, G_NCH, chunk, 0)


def _sc_gather_rows(table, idx_padded):
    mesh = plsc.VectorSubcoreMesh(core_axis_name="c", subcore_axis_name="s")
    return pl.kernel(
        _sc_gather_body,
        out_type=jax.ShapeDtypeStruct((G_TOT, NH), jnp.float32),
        mesh=mesh,
        scratch_types=[
            pltpu.VMEM((G_CHUNK,), jnp.int32),
            pltpu.VMEM((G_CHUNK, NH), jnp.float32),
            pltpu.SemaphoreType.DMA,
        ],
    )(table, idx_padded)


def _unsplit(out64):
    o = out64.reshape(2, 2, NP, HW)[:, :, :N, :]
    return jnp.concatenate([o[:, 0], o[:, 1]], axis=-1)


def _bn2(o, g, b):
    m = jnp.mean(o, axis=1, keepdims=True)
    v = jnp.var(o, axis=1, keepdims=True)
    return (o - m) / jnp.sqrt(v + 1e-5) * g + b


def _prelu(x, a):
    return jnp.where(x >= 0, x, a * x)


def _loss_body(num_ref, hisq_ref, htsq_ref, out_ref):
    num = num_ref[...]
    hisq = hisq_ref[...]
    htsq = htsq_ref[...]
    den = jnp.maximum(jnp.sqrt(hisq) * jnp.sqrt(htsq), EPS_COS)
    sim = num / den
    ex = jnp.exp(sim)
    numerator = ex[:, 0:1]
    denominator = jnp.sum(ex[:, 1:], axis=1, keepdims=True)
    res = -jnp.log(numerator / denominator)
    out_ref[...] = jnp.sum(res, axis=0, keepdims=True) / N


def kernel(seq, edge_index, edge_weight, samples, fea_w, str_w0, str_w1,
           W0, b0, a0, W1, b1, a1, g0, bnb0, g1, bnb1):
    x = jnp.squeeze(seq, axis=0)
    dst = edge_index[0]
    src = edge_index[1]
    pad_d = jnp.full((NS, EPW - EREAL), N, jnp.int32)
    pad_z = jnp.zeros((NS, EPW - EREAL), jnp.int32)
    dstp = jnp.concatenate([dst.reshape(NS, EREAL), pad_d], 1)
    srcp = jnp.concatenate([src.reshape(NS, EREAL), pad_z], 1)
    wp = jnp.concatenate(
        [lax.bitcast_convert_type(edge_weight, jnp.int32).reshape(NS, EREAL),
         pad_z], 1)
    edata = jnp.stack([dstp.reshape(NS * S_NCH, S_CHUNK),
                       srcp.reshape(NS * S_NCH, S_CHUNK),
                       wp.reshape(NS * S_NCH, S_CHUNK)], axis=1)

    W0pair = jnp.stack([fea_w[0][:, None] * W0, W0])
    h1 = jnp.einsum("nk,pkj->pnj", x, W0pair).reshape(4 * N, HW)
    o1 = _unsplit(_sc_spmm(h1, edata))
    o1 = _prelu(o1 + b0, a0)
    o1 = o1 * jnp.stack([jnp.ones((NH,), o1.dtype), str_w0[0]])[:, None, :]
    o1 = _bn2(o1, g0, bnb0)

    h2 = jnp.einsum("pnj,jk->pnk", o1, W1).reshape(4 * N, HW)
    o2 = _unsplit(_sc_spmm(h2, edata))
    o2 = _prelu(o2 + b1, a1)
    o2 = o2 * jnp.stack([jnp.ones((NH,), o2.dtype), str_w1[0]])[:, None, :]
    o2 = _bn2(o2, g1, bnb1)

    logits = jax.nn.elu(o2[0]) + jax.nn.elu(o2[1])

    idx_flat = jnp.pad(samples.reshape(-1), (0, G_TOT - 5 * N))
    h_t = _sc_gather_rows(logits, idx_flat)[: 5 * N].reshape(N, 5, NH)
    num = jnp.einsum("nd,nkd->nk", logits, h_t)
    hisq = jnp.sum(logits * logits, axis=1, keepdims=True)
    htsq = jnp.sum(h_t * h_t, axis=2)

    out = pl.pallas_call(
        _loss_body,
        out_shape=jax.ShapeDtypeStruct((1, 1), jnp.float32),
    )(num, hisq, htsq)
    return out[0, 0]
